# baseline (speedup 1.0000x reference)
import functools

import jax
import jax.numpy as jnp
from jax import lax
from jax.experimental import pallas as pl
from jax.experimental.pallas import tpu as pltpu
from jax.experimental.pallas import tpu_sc as plsc

CUT = 15
HID = 768
B = 256
S = 576

_info = plsc.get_sparse_core_info()
_NC, _NS = _info.num_cores, _info.num_subcores
_NW = _NC * _NS
_BPW = B // _NW


def _sc_lookup(h_table, w_table, idx0, idx1):
    mesh = plsc.VectorSubcoreMesh(core_axis_name="c", subcore_axis_name="s")

    @functools.partial(
        pl.kernel,
        mesh=mesh,
        out_type=jax.ShapeDtypeStruct((B, HID), jnp.float32),
        scratch_types=[
            pltpu.VMEM((_BPW,), jnp.int32),
            pltpu.VMEM((_BPW,), jnp.int32),
            pltpu.VMEM((_BPW, HID), jnp.float32),
            pltpu.VMEM((_BPW, HID), jnp.float32),
            pltpu.SemaphoreType.DMA,
            pltpu.SemaphoreType.DMA,
        ],
    )
    def k(h_hbm, w_hbm, i0_hbm, i1_hbm, pe_hbm,
          i0_v, i1_v, hr_v, wr_v, s0, s1):
        wid = lax.axis_index("s") * _NC + lax.axis_index("c")
        base = wid * _BPW
        pltpu.sync_copy(i0_hbm.at[pl.ds(base, _BPW)], i0_v)
        pltpu.sync_copy(i1_hbm.at[pl.ds(base, _BPW)], i1_v)
        c0 = pltpu.async_copy(h_hbm.at[i0_v], hr_v, s0)
        c1 = pltpu.async_copy(w_hbm.at[i1_v], wr_v, s1)
        c0.wait()
        c1.wait()
        for r in range(_BPW):
            def pe_add(j, _, r=r):
                sl = pl.ds(j * 16, 16)
                hr_v[r, sl] = hr_v[r, sl] + wr_v[r, sl]
                return 0
            lax.fori_loop(0, HID // 16, pe_add, 0, unroll=8)
        pltpu.sync_copy(hr_v, pe_hbm.at[pl.ds(base, _BPW)])

    return k(h_table, w_table, idx0, idx1)


_NBUF = 8
_CHB = 4
_NST = B // _CHB
_LAG = 4
_SJ = 48


def _tc_body(h_hbm, pe_ref, o_hbm, bufs, isems, osems):
    def in_cp(i, k):
        return pltpu.make_async_copy(
            h_hbm.at[pl.ds(i * _CHB, _CHB)], bufs.at[k], isems.at[k])

    def out_cp(i, k):
        return pltpu.make_async_copy(
            bufs.at[k], o_hbm.at[pl.ds(i * _CHB, _CHB)], osems.at[k])

    for k in range(_NBUF):
        in_cp(k, k).start()

    def step(m, _):
        k = lax.rem(m, _NBUF)
        mm = m - _LAG
        k2 = lax.rem(mm + _NBUF, _NBUF)

        @pl.when(m >= _LAG)
        def _():
            out_cp(mm, k2).wait()

        @pl.when(jnp.logical_and(m >= _LAG, mm + _NBUF < _NST))
        def _():
            in_cp(mm + _NBUF, k2).start()

        in_cp(m, k).wait()
        pe = pe_ref[m]

        def sub(j, _):
            js = pl.ds(pl.multiple_of(j * _SJ, 8), _SJ)
            bufs[k, :, js, :] = bufs[k, :, js, :] + pe[:, None, :]
            return 0
        lax.fori_loop(0, S // _SJ, sub, 0)

        out_cp(m, k).start()
        return 0

    lax.fori_loop(0, _NST, step, 0)

    for mm in range(_NST - _LAG, _NST):
        out_cp(mm, mm % _NBUF).wait()


def _tc_add(hidden, pe_rows):
    return pl.pallas_call(
        _tc_body,
        in_specs=[
            pl.BlockSpec(memory_space=pl.ANY),
            pl.BlockSpec(memory_space=pltpu.MemorySpace.VMEM),
        ],
        out_specs=pl.BlockSpec(memory_space=pl.ANY),
        out_shape=jax.ShapeDtypeStruct((B, S, HID), jnp.float32),
        scratch_shapes=[
            pltpu.VMEM((_NBUF, _CHB, S, HID), jnp.float32),
            pltpu.SemaphoreType.DMA((_NBUF,)),
            pltpu.SemaphoreType.DMA((_NBUF,)),
        ],
    )(hidden, pe_rows.reshape(_NST, _CHB, HID))


def kernel(hidden_states, patch_positions, h_table, w_table):
    idx0 = patch_positions[:, 0].astype(jnp.int32)
    idx1 = patch_positions[:, 1].astype(jnp.int32)
    pe_rows = _sc_lookup(h_table, w_table, idx0, idx1)
    return _tc_add(hidden_states, pe_rows)


# speedup vs baseline: 1.0175x; 1.0175x over previous
import functools

import jax
import jax.numpy as jnp
from jax import lax
from jax.experimental import pallas as pl
from jax.experimental.pallas import tpu as pltpu
from jax.experimental.pallas import tpu_sc as plsc

CUT = 15
HID = 768
B = 256
S = 576

_info = plsc.get_sparse_core_info()
_NC, _NS = _info.num_cores, _info.num_subcores
_NW = _NC * _NS
_BPW = B // _NW


def _sc_lookup(h_table, w_table, idx0, idx1):
    mesh = plsc.VectorSubcoreMesh(core_axis_name="c", subcore_axis_name="s")

    @functools.partial(
        pl.kernel,
        mesh=mesh,
        out_type=[
            jax.ShapeDtypeStruct((B, HID), jnp.float32),
            jax.ShapeDtypeStruct((B, HID), jnp.float32),
        ],
        scratch_types=[
            pltpu.VMEM((_BPW,), jnp.int32),
            pltpu.VMEM((_BPW,), jnp.int32),
            pltpu.VMEM((_BPW, HID), jnp.float32),
            pltpu.VMEM((_BPW, HID), jnp.float32),
            pltpu.SemaphoreType.DMA,
            pltpu.SemaphoreType.DMA,
        ],
    )
    def k(h_hbm, w_hbm, i0_hbm, i1_hbm, oh_hbm, ow_hbm,
          i0_v, i1_v, hr_v, wr_v, s0, s1):
        wid = lax.axis_index("s") * _NC + lax.axis_index("c")
        base = wid * _BPW
        pltpu.sync_copy(i0_hbm.at[pl.ds(base, _BPW)], i0_v)
        pltpu.sync_copy(i1_hbm.at[pl.ds(base, _BPW)], i1_v)
        c0 = pltpu.async_copy(h_hbm.at[i0_v], hr_v, s0)
        c1 = pltpu.async_copy(w_hbm.at[i1_v], wr_v, s1)
        c0.wait()
        c1.wait()
        pltpu.sync_copy(hr_v, oh_hbm.at[pl.ds(base, _BPW)])
        pltpu.sync_copy(wr_v, ow_hbm.at[pl.ds(base, _BPW)])

    return k(h_table, w_table, idx0, idx1)


_BB = 8


def _tc_body(h_ref, hr_ref, wr_ref, o_ref):
    base = pl.multiple_of(pl.program_id(0) * _BB, _BB)
    pe = hr_ref[pl.ds(base, _BB), :] + wr_ref[pl.ds(base, _BB), :]
    o_ref[...] = h_ref[...] + pe[:, None, :]


def _tc_add(hidden, h_rows, w_rows):
    return pl.pallas_call(
        _tc_body,
        grid=(B // _BB,),
        in_specs=[
            pl.BlockSpec((_BB, S, HID), lambda b: (b, 0, 0)),
            pl.BlockSpec((B, HID), lambda b: (0, 0)),
            pl.BlockSpec((B, HID), lambda b: (0, 0)),
        ],
        out_specs=pl.BlockSpec((_BB, S, HID), lambda b: (b, 0, 0)),
        out_shape=jax.ShapeDtypeStruct((B, S, HID), jnp.float32),
    )(hidden, h_rows, w_rows)


def kernel(hidden_states, patch_positions, h_table, w_table):
    idx0 = patch_positions[:, 0].astype(jnp.int32)
    idx1 = patch_positions[:, 1].astype(jnp.int32)
    h_rows, w_rows = _sc_lookup(h_table, w_table, idx0, idx1)
    return _tc_add(hidden_states, h_rows, w_rows)
